# Initial kernel scaffold; baseline (speedup 1.0000x reference)
import jax
import jax.numpy as jnp
from jax.experimental import pallas as pl


def kernel(mix, falsetto, breathy, pharyngeal, glissando, vibrato, emotion, singing_method, pace, range_, mix_W, falsetto_W, breathy_W, pharyngeal_W, glissando_W, vibrato_W, emotion_W, singing_method_W, pace_W, range_W):
    raise NotImplementedError("write your pallas kernel here")


# speedup vs baseline: 15.1034x; 15.1034x over previous
import math

import jax
import jax.numpy as jnp
from jax import lax
from jax.experimental import pallas as pl
from jax.experimental.pallas import tpu as pltpu

H = 1024
B, T = 4, 8192
SCALE = math.sqrt(H)
TB = 512
NB = (B * T) // TB
BLOCKS_PER_BATCH = T // TB


def _tc_body(em_sm, sm_sm, pc_sm, rg_sm,
             mix_r, fal_r, bre_r, pha_r, gli_r, vib_r,
             wstack_r, em_w, sm_w, pc_w, rg_w, out_r):
    i = pl.program_id(0)
    b = i // BLOCKS_PER_BATCH

    idxs = [r[0, 0, :] for r in (mix_r, fal_r, bre_r, pha_r, gli_r, vib_r)]
    cols = lax.broadcasted_iota(jnp.int32, (TB, 18), 1)
    k = cols // 3
    d = cols % 3
    sel = idxs[5][:, None]
    for kk in range(4, -1, -1):
        sel = jnp.where(k == kk, idxs[kk][:, None], sel)
    onehot = (sel == d).astype(jnp.float32)
    x = jnp.dot(onehot, wstack_r[...], preferred_element_type=jnp.float32)

    bias = em_w[pl.ds(em_sm[b], 1), :]
    bias = bias + sm_w[pl.ds(sm_sm[b], 1), :]
    bias = bias + pc_w[pl.ds(pc_sm[b], 1), :]
    bias = bias + rg_w[pl.ds(rg_sm[b], 1), :]
    out_r[...] = (x + bias) * SCALE


def kernel(mix, falsetto, breathy, pharyngeal, glissando, vibrato,
           emotion, singing_method, pace, range_,
           mix_W, falsetto_W, breathy_W, pharyngeal_W, glissando_W, vibrato_W,
           emotion_W, singing_method_W, pace_W, range_W):
    wstack = jnp.concatenate(
        [mix_W, falsetto_W, breathy_W, pharyngeal_W, glissando_W, vibrato_W],
        axis=0)
    seq3d = [a.reshape(NB, 1, TB) for a in
             (mix, falsetto, breathy, pharyngeal, glissando, vibrato)]

    idx_spec = pl.BlockSpec((1, 1, TB), lambda i: (i, 0, 0))
    full = lambda s: pl.BlockSpec(s, lambda i: (0,) * len(s))
    smem = pl.BlockSpec(memory_space=pltpu.SMEM)

    out = pl.pallas_call(
        _tc_body,
        grid=(NB,),
        in_specs=[smem, smem, smem, smem,
                  idx_spec, idx_spec, idx_spec, idx_spec, idx_spec, idx_spec,
                  full((18, H)), full((4, H)), full((4, H)), full((5, H)),
                  full((5, H))],
        out_specs=pl.BlockSpec((TB, H), lambda i: (i, 0)),
        out_shape=jax.ShapeDtypeStruct((B * T, H), jnp.float32),
    )(emotion, singing_method, pace, range_, *seq3d,
      wstack, emotion_W, singing_method_W, pace_W, range_W)
    return out.reshape(B, T, H)
